# baseline (speedup 1.0000x reference)
import functools

import jax
import jax.numpy as jnp
from jax import lax
from jax.experimental import pallas as pl
from jax.experimental.pallas import tpu as pltpu
from jax.experimental.pallas import tpu_sc as plsc

_NC, _NS, _L = 2, 16, 16
_NW = _NC * _NS
_UNROLL = 8


@functools.lru_cache(maxsize=None)
def _make_sc(B, S, D):
    s_per_w = S // _NW
    CH = 32
    n_ch = s_per_w // CH
    pe_words = s_per_w * D
    ch_words = CH * D

    mesh = plsc.VectorSubcoreMesh(
        core_axis_name="c", subcore_axis_name="s",
        num_cores=_NC, num_subcores=_NS)

    @functools.partial(
        pl.kernel, mesh=mesh,
        out_type=jax.ShapeDtypeStruct((B * S * D,), jnp.float32),
        scratch_types=[
            pltpu.VMEM((pe_words,), jnp.float32),
            pltpu.VMEM((ch_words,), jnp.float32),
        ],
    )
    def k(x_hbm, pe_hbm, out_hbm, pe_v, x_v):
        wid = lax.axis_index("s") * _NC + lax.axis_index("c")
        base = wid * pe_words
        pltpu.sync_copy(pe_hbm.at[pl.ds(base, pe_words)], pe_v)
        for j in range(n_ch):
            pe_off = j * ch_words
            for b in range(B):
                off = b * S * D + base + pe_off
                pltpu.sync_copy(x_hbm.at[pl.ds(off, ch_words)], x_v)

                def body(i, _, pe_off=pe_off):
                    i0 = i * (_L * _UNROLL)
                    for u in range(_UNROLL):
                        sl = i0 + u * _L
                        plsc.addupdate(
                            x_v.at[pl.ds(sl, _L)],
                            pe_v[pl.ds(pe_off + sl, _L)])
                    return 0

                lax.fori_loop(0, ch_words // (_L * _UNROLL), body, 0)
                pltpu.sync_copy(x_v, out_hbm.at[pl.ds(off, ch_words)])

    return k


def kernel(inputs, pos_embed):
    B, S, D = inputs.shape
    x = inputs.reshape(-1)
    pe = pos_embed.reshape(-1)
    out = _make_sc(B, S, D)(x, pe)
    return out.reshape(B, S, D)


# speedup vs baseline: 1.1114x; 1.1114x over previous
import functools

import jax
import jax.numpy as jnp
from jax import lax
from jax.experimental import pallas as pl
from jax.experimental.pallas import tpu as pltpu
from jax.experimental.pallas import tpu_sc as plsc

_NC, _NS, _L = 2, 16, 16
_NW = _NC * _NS
_UNROLL = 8
_CH = 16


@functools.lru_cache(maxsize=None)
def _make_sc(B, S, D):
    s_per_w = S // _NW
    n_ch = s_per_w // _CH
    n_chunks = n_ch * B
    pe_words = s_per_w * D
    ch_words = _CH * D

    mesh = plsc.VectorSubcoreMesh(
        core_axis_name="c", subcore_axis_name="s",
        num_cores=_NC, num_subcores=_NS)

    @functools.partial(
        pl.kernel, mesh=mesh,
        out_type=jax.ShapeDtypeStruct((B * S * D,), jnp.float32),
        scratch_types=[
            pltpu.VMEM((pe_words,), jnp.float32),
            pltpu.VMEM((ch_words,), jnp.float32),
            pltpu.VMEM((ch_words,), jnp.float32),
            pltpu.SemaphoreType.DMA,
            pltpu.SemaphoreType.DMA,
            pltpu.SemaphoreType.DMA,
            pltpu.SemaphoreType.DMA,
            pltpu.SemaphoreType.DMA,
        ],
    )
    def k(x_hbm, pe_hbm, out_hbm, pe_v, x_v0, x_v1,
          pe_sem, in_s0, in_s1, out_s0, out_s1):
        wid = lax.axis_index("s") * _NC + lax.axis_index("c")
        base = wid * pe_words
        xv = (x_v0, x_v1)
        ins = (in_s0, in_s1)
        outs = (out_s0, out_s1)

        pe_d = pltpu.async_copy(
            pe_hbm.at[pl.ds(base, pe_words)], pe_v, pe_sem)

        def off(t):
            j, b = divmod(t, B)
            return b * S * D + base + j * ch_words

        def start_in(t):
            return pltpu.async_copy(
                x_hbm.at[pl.ds(off(t), ch_words)], xv[t % 2], ins[t % 2])

        def start_out(t):
            return pltpu.async_copy(
                xv[t % 2], out_hbm.at[pl.ds(off(t), ch_words)], outs[t % 2])

        d_in = {0: start_in(0)}
        d_out = {}
        pe_d.wait()
        for t in range(n_chunks):
            if t + 1 < n_chunks:
                if t >= 1:
                    d_out[t - 1].wait()
                d_in[t + 1] = start_in(t + 1)
            d_in[t].wait()
            pe_off = (t // B) * ch_words
            buf = xv[t % 2]

            def body(i, buf=buf, pe_off=pe_off):
                plsc.addupdate(
                    buf.at[pl.ds(i, _L)], pe_v[pl.ds(pe_off + i, _L)])

            plsc.parallel_loop(0, ch_words, _L, unroll=_UNROLL)(body)
            d_out[t] = start_out(t)
        d_out[n_chunks - 2].wait()
        d_out[n_chunks - 1].wait()

    return k


def kernel(inputs, pos_embed):
    B, S, D = inputs.shape
    x = inputs.reshape(-1)
    pe = pos_embed.reshape(-1)
    out = _make_sc(B, S, D)(x, pe)
    return out.reshape(B, S, D)


# speedup vs baseline: 2.7520x; 2.4760x over previous
import functools

import jax
import jax.numpy as jnp
from jax import lax
from jax.experimental import pallas as pl
from jax.experimental.pallas import tpu as pltpu
from jax.experimental.pallas import tpu_sc as plsc

_NC, _NS, _L = 2, 16, 16
_NW = _NC * _NS
_UNROLL = 8
_CH = 16


@functools.lru_cache(maxsize=None)
def _make_sc(B, S, D):
    s_per_w = S // _NW
    n_ch = s_per_w // _CH
    n_chunks = n_ch * B

    mesh = plsc.VectorSubcoreMesh(
        core_axis_name="c", subcore_axis_name="s",
        num_cores=_NC, num_subcores=_NS)

    @functools.partial(
        pl.kernel, mesh=mesh,
        out_type=jax.ShapeDtypeStruct((B, S, D), jnp.float32),
        scratch_types=[
            pltpu.VMEM((s_per_w, D), jnp.float32),
            pltpu.VMEM((_CH, D), jnp.float32),
            pltpu.VMEM((_CH, D), jnp.float32),
            pltpu.SemaphoreType.DMA,
            pltpu.SemaphoreType.DMA,
            pltpu.SemaphoreType.DMA,
            pltpu.SemaphoreType.DMA,
            pltpu.SemaphoreType.DMA,
        ],
    )
    def k(x_hbm, pe_hbm, out_hbm, pe_v, x_v0, x_v1,
          pe_sem, in_s0, in_s1, out_s0, out_s1):
        wid = lax.axis_index("s") * _NC + lax.axis_index("c")
        base_s = wid * s_per_w
        xv = (x_v0, x_v1)
        ins = (in_s0, in_s1)
        outs = (out_s0, out_s1)

        pe_d = pltpu.async_copy(
            pe_hbm.at[pl.ds(base_s, s_per_w)], pe_v, pe_sem)

        def start_in(t):
            j, b = divmod(t, B)
            return pltpu.async_copy(
                x_hbm.at[b, pl.ds(base_s + j * _CH, _CH)],
                xv[t % 2], ins[t % 2])

        def start_out(t):
            j, b = divmod(t, B)
            return pltpu.async_copy(
                xv[t % 2],
                out_hbm.at[b, pl.ds(base_s + j * _CH, _CH)],
                outs[t % 2])

        d_in = {0: start_in(0)}
        d_out = {}
        pe_d.wait()
        for t in range(n_chunks):
            if t + 1 < n_chunks:
                if t >= 1:
                    d_out[t - 1].wait()
                d_in[t + 1] = start_in(t + 1)
            d_in[t].wait()
            j = t // B
            buf = xv[t % 2]
            dshift = D.bit_length() - 1

            def body(i, buf=buf, j=j):
                r = i >> dshift
                c = pl.multiple_of(i & (D - 1), _L)
                plsc.addupdate(
                    buf.at[r, pl.ds(c, _L)],
                    pe_v[j * _CH + r, pl.ds(c, _L)])

            plsc.parallel_loop(0, _CH * D, _L, unroll=_UNROLL)(body)
            d_out[t] = start_out(t)
        d_out[n_chunks - 2].wait()
        d_out[n_chunks - 1].wait()

    return k


def kernel(inputs, pos_embed):
    B, S, D = inputs.shape
    return _make_sc(B, S, D)(inputs, pos_embed)
